# baseline (speedup 1.0000x reference)
import jax
import jax.numpy as jnp
from jax import lax
from jax.experimental import pallas as pl
from jax.experimental.pallas import tpu as pltpu
from jax.experimental.pallas import tpu_sc as plsc

N_NODES = 10000
N_EDGES = 160000
D = 256
V = 21
LB = 32
NC, NS = 2, 16
NW = NC * NS
EPW = N_EDGES // NW
VPW = (EPW + 15) // 16 * 16
ROWS = (EPW + 127) // 128
EPAD = ROWS * 128
BSZ = N_NODES * LB
WPT = BSZ // NS

BE = 2000
GE = N_EDGES // BE
BN = 2000


def _sc_gather_body(S_hbm, src_hbm, dst_hbm, ssrc_hbm, sdst_hbm, sidx_hbm,
                    S_v, src_v, dst_v, ssrc_v, sdst_v, sidx_v):
    c = lax.axis_index("c")
    s = lax.axis_index("s")
    base = (s * NC + c) * EPW
    pltpu.sync_copy(S_hbm, S_v)
    pltpu.sync_copy(src_hbm.at[pl.ds(base, EPW)], src_v.at[pl.ds(0, EPW)])
    pltpu.sync_copy(dst_hbm.at[pl.ds(base, EPW)], dst_v.at[pl.ds(0, EPW)])

    def body(i, carry):
        p = i * 16
        sv = src_v[pl.ds(p, 16)]
        dv = dst_v[pl.ds(p, 16)]
        sv = jnp.minimum(jnp.maximum(sv, 0), N_NODES - 1)
        dv = jnp.minimum(jnp.maximum(dv, 0), N_NODES - 1)
        a = plsc.load_gather(S_v, [sv])
        b = plsc.load_gather(S_v, [dv])
        ssrc_v[pl.ds(p, 16)] = a
        sdst_v[pl.ds(p, 16)] = b
        sidx_v[pl.ds(p, 16)] = dv * LB + a
        return carry

    lax.fori_loop(0, VPW // 16, body, 0)
    pltpu.sync_copy(ssrc_v.at[pl.ds(0, EPW)], ssrc_hbm.at[pl.ds(base, EPW)])
    pltpu.sync_copy(sdst_v.at[pl.ds(0, EPW)], sdst_hbm.at[pl.ds(base, EPW)])
    pltpu.sync_copy(sidx_v.at[pl.ds(0, EPW)], sidx_hbm.at[pl.ds(base, EPW)])


def _sc_gather(S, src, dst):
    mesh = plsc.VectorSubcoreMesh(core_axis_name="c", subcore_axis_name="s")
    return pl.kernel(
        _sc_gather_body,
        out_type=[jax.ShapeDtypeStruct((N_EDGES,), jnp.int32)] * 3,
        mesh=mesh,
        compiler_params=pltpu.CompilerParams(needs_layout_passes=False),
        scratch_types=[
            pltpu.VMEM((N_NODES,), jnp.int32),
            pltpu.VMEM((VPW,), jnp.int32),
            pltpu.VMEM((VPW,), jnp.int32),
            pltpu.VMEM((VPW,), jnp.int32),
            pltpu.VMEM((VPW,), jnp.int32),
            pltpu.VMEM((VPW,), jnp.int32),
        ],
    )(S, src, dst)


def _sc_hist_body(att_hbm, sidx_hbm, bmax_hbm, out_hbm,
                  att_v, si_v, vv, iv2, bm_v, zb_v, B_sh):
    c = lax.axis_index("c")
    s = lax.axis_index("s")
    base = (s * NC + c) * EPW

    def zbody(i, carry):
        zb_v[pl.ds(i * 16, 16)] = jnp.zeros((16,), jnp.float32)
        return carry

    lax.fori_loop(0, WPT // 16, zbody, 0)
    pltpu.sync_copy(zb_v, B_sh.at[pl.ds(s * WPT, WPT)])
    plsc.subcore_barrier()

    pltpu.sync_copy(att_hbm.at[pl.ds(base, EPW)], att_v.at[pl.ds(0, EPW)])
    pltpu.sync_copy(sidx_hbm.at[pl.ds(base, EPW)], si_v.at[pl.ds(0, EPW)])
    pltpu.sync_copy(bmax_hbm, bm_v)
    m = bm_v[pl.ds(0, 16)]
    for k in range(1, GE // 16):
        m = jnp.maximum(m, bm_v[pl.ds(k * 16, 16)])
    g = lax.reduce_max(m, axes=(0,))
    lane = lax.iota(jnp.int32, 16)

    def body(i, carry):
        p = i * 16
        valid = (p + lane) < EPW
        e = jnp.exp(att_v[pl.ds(p, 16)] - g)
        e = jnp.where(valid, e, 0.0)
        ix = si_v[pl.ds(p, 16)]
        ix = jnp.where(valid, ix, 0)
        ix = jnp.minimum(jnp.maximum(ix, 0), BSZ - 1)
        vv[pl.ds(p, 16)] = e
        iv2[i // 8, pl.ds((i % 8) * 16, 16)] = ix
        return carry

    lax.fori_loop(0, EPAD // 16, body, 0)

    def srow(j, carry):
        pltpu.sync_copy(vv.at[pl.ds(j * 128, 128)], B_sh.at[iv2.at[j]],
                        add=True)
        return carry

    lax.fori_loop(0, ROWS, srow, 0)
    plsc.subcore_barrier()

    pltpu.sync_copy(B_sh.at[pl.ds(s * WPT, WPT)], zb_v)
    pltpu.sync_copy(zb_v, out_hbm.at[pl.ds(c * BSZ + s * WPT, WPT)])


def _sc_hist(att, sidx, bmax):
    mesh = plsc.VectorSubcoreMesh(core_axis_name="c", subcore_axis_name="s")
    return pl.kernel(
        _sc_hist_body,
        out_type=[jax.ShapeDtypeStruct((NC * BSZ,), jnp.float32)],
        mesh=mesh,
        compiler_params=pltpu.CompilerParams(needs_layout_passes=False),
        scratch_types=[
            pltpu.VMEM((EPAD,), jnp.float32),
            pltpu.VMEM((EPAD,), jnp.int32),
            pltpu.VMEM((EPAD,), jnp.float32),
            pltpu.VMEM((ROWS, 128), jnp.int32),
            pltpu.VMEM((GE,), jnp.float32),
            pltpu.VMEM((WPT,), jnp.float32),
            pltpu.VMEM_SHARED((BSZ,), jnp.float32),
        ],
    )(att, sidx, bmax)


def _att_body(ea_ref, ss_ref, sd_ref, we_ref, ws_ref, wsb_ref, wdb_ref,
              a2_ref, att_ref, bmax_ref):
    f32 = jnp.float32
    ts = jnp.dot(ws_ref[...], wsb_ref[...], preferred_element_type=f32)
    td = jnp.dot(ws_ref[...], wdb_ref[...], preferred_element_type=f32)
    u = jnp.dot(ea_ref[...], we_ref[...], preferred_element_type=f32)
    iot = lax.broadcasted_iota(jnp.int32, (LB, 1), 0)
    ohsT = (ss_ref[0] == iot).astype(f32)
    ohdT = (sd_ref[0] == iot).astype(f32)
    dn = (((0,), (0,)), ((), ()))
    z = (u
         + lax.dot_general(ohsT, ts, dn, preferred_element_type=f32)
         + lax.dot_general(ohdT, td, dn, preferred_element_type=f32))
    zl = jnp.where(z > 0, z, 0.2 * z)
    att = jnp.sum(zl * a2_ref[...], axis=1)
    att_ref[...] = att.reshape(1, 1, BE)
    bmax_ref[...] = jnp.broadcast_to(jnp.max(att), (1, 1, 1))


def _out_body(bp_ref, ws_ref, wsb_ref, wo_ref, b_ref, out_ref):
    f32 = jnp.float32
    ts = jnp.dot(ws_ref[...], wsb_ref[...], preferred_element_type=f32)
    m = jnp.dot(ts, wo_ref[...], preferred_element_type=f32)
    bs = bp_ref[0] + bp_ref[1]
    denom = jnp.sum(bs, axis=1, keepdims=True)
    a = bs / (denom + 1e-16)
    logits = jnp.dot(a, m, preferred_element_type=f32) + b_ref[...]
    col = lax.broadcasted_iota(jnp.int32, (1, 128), 1)
    valid = col < V
    lm = jnp.where(valid, logits, -1e30)
    mx = jnp.max(lm, axis=1, keepdims=True)
    ex = jnp.where(valid, jnp.exp(logits - mx), 0.0)
    lse = mx + jnp.log(jnp.sum(ex, axis=1, keepdims=True))
    out_ref[...] = (logits - lse)[:, :V]


def kernel(S, edge_index, edge_attr, W_src1, W_dst1, W_e1, a1, W_s,
           W_src2, W_dst2, W_e2, a2, W_out, b_out):
    f32 = jnp.float32
    S = S.astype(jnp.int32)
    src = edge_index[0].astype(jnp.int32)
    dst = edge_index[1].astype(jnp.int32)

    ws_pad = jnp.zeros((LB, D), f32).at[:V].set(W_s)
    wo_pad = jnp.zeros((D, 128), f32).at[:, :V].set(W_out)
    b_pad = jnp.zeros((1, 128), f32).at[0, :V].set(b_out)

    ssrc, sdst, sidx = _sc_gather(S, src, dst)

    att3, bmax3 = pl.pallas_call(
        _att_body,
        grid=(GE,),
        in_specs=[
            pl.BlockSpec((BE, 128), lambda i: (i, 0)),
            pl.BlockSpec((1, 1, BE), lambda i: (i, 0, 0)),
            pl.BlockSpec((1, 1, BE), lambda i: (i, 0, 0)),
            pl.BlockSpec((128, D), lambda i: (0, 0)),
            pl.BlockSpec((LB, D), lambda i: (0, 0)),
            pl.BlockSpec((D, D), lambda i: (0, 0)),
            pl.BlockSpec((D, D), lambda i: (0, 0)),
            pl.BlockSpec((1, D), lambda i: (0, 0)),
        ],
        out_specs=[
            pl.BlockSpec((1, 1, BE), lambda i: (i, 0, 0)),
            pl.BlockSpec((1, 1, 1), lambda i: (i, 0, 0)),
        ],
        out_shape=[
            jax.ShapeDtypeStruct((GE, 1, BE), f32),
            jax.ShapeDtypeStruct((GE, 1, 1), f32),
        ],
    )(edge_attr, ssrc.reshape(GE, 1, BE), sdst.reshape(GE, 1, BE),
      W_e2, ws_pad, W_src2[D:], W_dst2[D:], a2.reshape(1, D))

    (bp,) = _sc_hist(att3.reshape(N_EDGES), sidx, bmax3.reshape(GE))

    out = pl.pallas_call(
        _out_body,
        grid=(N_NODES // BN,),
        in_specs=[
            pl.BlockSpec((NC, BN, LB), lambda i: (0, i, 0)),
            pl.BlockSpec((LB, D), lambda i: (0, 0)),
            pl.BlockSpec((D, D), lambda i: (0, 0)),
            pl.BlockSpec((D, 128), lambda i: (0, 0)),
            pl.BlockSpec((1, 128), lambda i: (0, 0)),
        ],
        out_specs=pl.BlockSpec((BN, V), lambda i: (i, 0)),
        out_shape=jax.ShapeDtypeStruct((N_NODES, V), f32),
    )(bp.reshape(NC, N_NODES, LB), ws_pad, W_src2[D:], wo_pad, b_pad)

    return out


# speedup vs baseline: 2.0308x; 2.0308x over previous
import jax
import jax.numpy as jnp
from jax import lax
from jax.experimental import pallas as pl
from jax.experimental.pallas import tpu as pltpu
from jax.experimental.pallas import tpu_sc as plsc

N_NODES = 10000
N_EDGES = 160000
D = 256
V = 21
LB = 32
NC, NS = 2, 16
NW = NC * NS
EPW = N_EDGES // NW
VPW = (EPW + 15) // 16 * 16
ROWS = (EPW + 127) // 128
EPAD = ROWS * 128
BSZ = N_NODES * LB
WPT = BSZ // NS

BE = 2000
GE = N_EDGES // BE
BN = 2000


def _sc_gather_body(S_hbm, src_hbm, dst_hbm, ssrc_hbm, sdst_hbm, sidx_hbm,
                    S_v, src_v, dst_v, ssrc_v, sdst_v, sidx_v):
    c = lax.axis_index("c")
    s = lax.axis_index("s")
    base = (s * NC + c) * EPW
    pltpu.sync_copy(S_hbm, S_v)
    pltpu.sync_copy(src_hbm.at[pl.ds(base, EPW)], src_v.at[pl.ds(0, EPW)])
    pltpu.sync_copy(dst_hbm.at[pl.ds(base, EPW)], dst_v.at[pl.ds(0, EPW)])

    def body(i, carry):
        p = i * 16
        sv = src_v[pl.ds(p, 16)]
        dv = dst_v[pl.ds(p, 16)]
        sv = jnp.minimum(jnp.maximum(sv, 0), N_NODES - 1)
        dv = jnp.minimum(jnp.maximum(dv, 0), N_NODES - 1)
        a = plsc.load_gather(S_v, [sv])
        b = plsc.load_gather(S_v, [dv])
        ssrc_v[pl.ds(p, 16)] = a
        sdst_v[pl.ds(p, 16)] = b
        sidx_v[pl.ds(p, 16)] = dv * LB + a
        return carry

    lax.fori_loop(0, VPW // 16, body, 0)
    pltpu.sync_copy(ssrc_v.at[pl.ds(0, EPW)], ssrc_hbm.at[pl.ds(base, EPW)])
    pltpu.sync_copy(sdst_v.at[pl.ds(0, EPW)], sdst_hbm.at[pl.ds(base, EPW)])
    pltpu.sync_copy(sidx_v.at[pl.ds(0, EPW)], sidx_hbm.at[pl.ds(base, EPW)])


def _sc_gather(S, src, dst):
    mesh = plsc.VectorSubcoreMesh(core_axis_name="c", subcore_axis_name="s")
    return pl.kernel(
        _sc_gather_body,
        out_type=[jax.ShapeDtypeStruct((N_EDGES,), jnp.int32)] * 3,
        mesh=mesh,
        compiler_params=pltpu.CompilerParams(needs_layout_passes=False),
        scratch_types=[
            pltpu.VMEM((N_NODES,), jnp.int32),
            pltpu.VMEM((VPW,), jnp.int32),
            pltpu.VMEM((VPW,), jnp.int32),
            pltpu.VMEM((VPW,), jnp.int32),
            pltpu.VMEM((VPW,), jnp.int32),
            pltpu.VMEM((VPW,), jnp.int32),
        ],
    )(S, src, dst)


def _sc_hist_body(att_hbm, sidx_hbm, bmax_hbm, out_hbm,
                  att_v, si_v, vv, iv2, bm_v, zb_v, B_sh):
    c = lax.axis_index("c")
    s = lax.axis_index("s")
    base = (s * NC + c) * EPW

    def zbody(i, carry):
        zb_v[pl.ds(i * 16, 16)] = jnp.zeros((16,), jnp.float32)
        return carry

    lax.fori_loop(0, WPT // 16, zbody, 0)
    pltpu.sync_copy(zb_v, B_sh.at[pl.ds(s * WPT, WPT)])
    plsc.subcore_barrier()

    pltpu.sync_copy(att_hbm.at[pl.ds(base, EPW)], att_v.at[pl.ds(0, EPW)])
    pltpu.sync_copy(sidx_hbm.at[pl.ds(base, EPW)], si_v.at[pl.ds(0, EPW)])
    pltpu.sync_copy(bmax_hbm, bm_v)
    m = bm_v[pl.ds(0, 16)]
    for k in range(1, GE // 16):
        m = jnp.maximum(m, bm_v[pl.ds(k * 16, 16)])
    g = lax.reduce_max(m, axes=(0,))
    lane = lax.iota(jnp.int32, 16)

    def body(i, carry):
        p = i * 16
        valid = (p + lane) < EPW
        e = jnp.exp(att_v[pl.ds(p, 16)] - g)
        e = jnp.where(valid, e, 0.0)
        ix = si_v[pl.ds(p, 16)]
        ix = jnp.where(valid, ix, 0)
        ix = jnp.minimum(jnp.maximum(ix, 0), BSZ - 1)
        vv[pl.ds(p, 16)] = e
        iv2[i // 8, pl.ds((i % 8) * 16, 16)] = ix
        return carry

    lax.fori_loop(0, EPAD // 16, body, 0)

    def srow(j, carry):
        pltpu.sync_copy(vv.at[pl.ds(j * 128, 128)], B_sh.at[iv2.at[j]],
                        add=True)
        return carry

    lax.fori_loop(0, ROWS, srow, 0)
    plsc.subcore_barrier()

    pltpu.sync_copy(B_sh.at[pl.ds(s * WPT, WPT)], zb_v)
    pltpu.sync_copy(zb_v, out_hbm.at[pl.ds(c * BSZ + s * WPT, WPT)])


def _sc_hist(att, sidx, bmax):
    mesh = plsc.VectorSubcoreMesh(core_axis_name="c", subcore_axis_name="s")
    return pl.kernel(
        _sc_hist_body,
        out_type=[jax.ShapeDtypeStruct((NC * BSZ,), jnp.float32)],
        mesh=mesh,
        compiler_params=pltpu.CompilerParams(needs_layout_passes=False),
        scratch_types=[
            pltpu.VMEM((EPAD,), jnp.float32),
            pltpu.VMEM((EPAD,), jnp.int32),
            pltpu.VMEM((EPAD,), jnp.float32),
            pltpu.VMEM((ROWS, 128), jnp.int32),
            pltpu.VMEM((GE,), jnp.float32),
            pltpu.VMEM((WPT,), jnp.float32),
            pltpu.VMEM_SHARED((BSZ,), jnp.float32),
        ],
    )(att, sidx, bmax)


def _att_body(ea_ref, ss_ref, sd_ref, we_ref, ws_ref, wsb_ref, wdb_ref,
              a2_ref, att_ref, bmax_ref):
    f32 = jnp.float32
    bf = jnp.bfloat16
    ts = jnp.dot(ws_ref[...], wsb_ref[...],
                 preferred_element_type=f32).astype(bf)
    td = jnp.dot(ws_ref[...], wdb_ref[...],
                 preferred_element_type=f32).astype(bf)
    u = jnp.dot(ea_ref[...].astype(bf), we_ref[...].astype(bf),
                preferred_element_type=f32)
    iot = lax.broadcasted_iota(jnp.int32, (LB, 1), 0)
    ohsT = (ss_ref[0] == iot).astype(bf)
    ohdT = (sd_ref[0] == iot).astype(bf)
    ohb = jnp.concatenate([ohsT, ohdT], axis=0)
    tsd = jnp.concatenate([ts, td], axis=0)
    dn = (((0,), (0,)), ((), ()))
    z = u + lax.dot_general(ohb, tsd, dn, preferred_element_type=f32)
    zl = jnp.maximum(z, 0.2 * z)
    att = lax.dot_general(a2_ref[...], zl, (((1,), (1,)), ((), ())),
                          preferred_element_type=f32)
    att_ref[...] = att[None]
    bmax_ref[...] = jnp.broadcast_to(jnp.max(att), (1, 1, 1))


def _out_body(bp_ref, ws_ref, wsb_ref, wo_ref, b_ref, out_ref):
    f32 = jnp.float32
    ts = jnp.dot(ws_ref[...], wsb_ref[...], preferred_element_type=f32)
    m = jnp.dot(ts, wo_ref[...], preferred_element_type=f32)
    bs = bp_ref[0] + bp_ref[1]
    denom = jnp.sum(bs, axis=1, keepdims=True)
    a = bs / (denom + 1e-16)
    logits = jnp.dot(a, m, preferred_element_type=f32) + b_ref[...]
    col = lax.broadcasted_iota(jnp.int32, (1, 128), 1)
    valid = col < V
    lm = jnp.where(valid, logits, -1e30)
    mx = jnp.max(lm, axis=1, keepdims=True)
    ex = jnp.where(valid, jnp.exp(logits - mx), 0.0)
    lse = mx + jnp.log(jnp.sum(ex, axis=1, keepdims=True))
    out_ref[...] = (logits - lse)[:, :V]


def kernel(S, edge_index, edge_attr, W_src1, W_dst1, W_e1, a1, W_s,
           W_src2, W_dst2, W_e2, a2, W_out, b_out):
    f32 = jnp.float32
    S = S.astype(jnp.int32)
    src = edge_index[0].astype(jnp.int32)
    dst = edge_index[1].astype(jnp.int32)

    ws_pad = jnp.zeros((LB, D), f32).at[:V].set(W_s)
    wo_pad = jnp.zeros((D, 128), f32).at[:, :V].set(W_out)
    b_pad = jnp.zeros((1, 128), f32).at[0, :V].set(b_out)

    ssrc, sdst, sidx = _sc_gather(S, src, dst)

    att3, bmax3 = pl.pallas_call(
        _att_body,
        grid=(GE,),
        in_specs=[
            pl.BlockSpec((BE, 128), lambda i: (i, 0)),
            pl.BlockSpec((1, 1, BE), lambda i: (i, 0, 0)),
            pl.BlockSpec((1, 1, BE), lambda i: (i, 0, 0)),
            pl.BlockSpec((128, D), lambda i: (0, 0)),
            pl.BlockSpec((LB, D), lambda i: (0, 0)),
            pl.BlockSpec((D, D), lambda i: (0, 0)),
            pl.BlockSpec((D, D), lambda i: (0, 0)),
            pl.BlockSpec((1, D), lambda i: (0, 0)),
        ],
        out_specs=[
            pl.BlockSpec((1, 1, BE), lambda i: (i, 0, 0)),
            pl.BlockSpec((1, 1, 1), lambda i: (i, 0, 0)),
        ],
        out_shape=[
            jax.ShapeDtypeStruct((GE, 1, BE), f32),
            jax.ShapeDtypeStruct((GE, 1, 1), f32),
        ],
    )(edge_attr, ssrc.reshape(GE, 1, BE), sdst.reshape(GE, 1, BE),
      W_e2, ws_pad, W_src2[D:], W_dst2[D:], a2.reshape(1, D))

    (bp,) = _sc_hist(att3.reshape(N_EDGES), sidx, bmax3.reshape(GE))

    out = pl.pallas_call(
        _out_body,
        grid=(N_NODES // BN,),
        in_specs=[
            pl.BlockSpec((NC, BN, LB), lambda i: (0, i, 0)),
            pl.BlockSpec((LB, D), lambda i: (0, 0)),
            pl.BlockSpec((D, D), lambda i: (0, 0)),
            pl.BlockSpec((D, 128), lambda i: (0, 0)),
            pl.BlockSpec((1, 128), lambda i: (0, 0)),
        ],
        out_specs=pl.BlockSpec((BN, V), lambda i: (i, 0)),
        out_shape=jax.ShapeDtypeStruct((N_NODES, V), f32),
    )(bp.reshape(NC, N_NODES, LB), ws_pad, W_src2[D:], wo_pad, b_pad)

    return out


# speedup vs baseline: 2.3427x; 1.1536x over previous
import jax
import jax.numpy as jnp
from jax import lax
from jax.experimental import pallas as pl
from jax.experimental.pallas import tpu as pltpu
from jax.experimental.pallas import tpu_sc as plsc

N_NODES = 10000
N_EDGES = 160000
D = 256
V = 21
LB = 32
NC, NS = 2, 16
NW = NC * NS
EPW = N_EDGES // NW
VPW = (EPW + 15) // 16 * 16
ROWS = (EPW + 127) // 128
EPAD = ROWS * 128
BSZ = N_NODES * LB
WPT = BSZ // NS

BE = 5000
GE = N_EDGES // BE
BN = 2000


def _sc_gather_body(S_hbm, src_hbm, dst_hbm, ssrc_hbm, sdst_hbm, sidx_hbm,
                    S_v, src_v, dst_v, ssrc_v, sdst_v, sidx_v):
    c = lax.axis_index("c")
    s = lax.axis_index("s")
    base = (s * NC + c) * EPW
    pltpu.sync_copy(S_hbm, S_v)
    pltpu.sync_copy(src_hbm.at[pl.ds(base, EPW)], src_v.at[pl.ds(0, EPW)])
    pltpu.sync_copy(dst_hbm.at[pl.ds(base, EPW)], dst_v.at[pl.ds(0, EPW)])

    def body(i, carry):
        p = i * 16
        sv = src_v[pl.ds(p, 16)]
        dv = dst_v[pl.ds(p, 16)]
        sv = jnp.minimum(jnp.maximum(sv, 0), N_NODES - 1)
        dv = jnp.minimum(jnp.maximum(dv, 0), N_NODES - 1)
        a = plsc.load_gather(S_v, [sv])
        b = plsc.load_gather(S_v, [dv])
        ssrc_v[pl.ds(p, 16)] = a
        sdst_v[pl.ds(p, 16)] = b
        sidx_v[pl.ds(p, 16)] = dv * LB + a
        return carry

    lax.fori_loop(0, VPW // 16, body, 0)
    pltpu.sync_copy(ssrc_v.at[pl.ds(0, EPW)], ssrc_hbm.at[pl.ds(base, EPW)])
    pltpu.sync_copy(sdst_v.at[pl.ds(0, EPW)], sdst_hbm.at[pl.ds(base, EPW)])
    pltpu.sync_copy(sidx_v.at[pl.ds(0, EPW)], sidx_hbm.at[pl.ds(base, EPW)])


def _sc_gather(S, src, dst):
    mesh = plsc.VectorSubcoreMesh(core_axis_name="c", subcore_axis_name="s")
    return pl.kernel(
        _sc_gather_body,
        out_type=[jax.ShapeDtypeStruct((N_EDGES,), jnp.int32)] * 3,
        mesh=mesh,
        compiler_params=pltpu.CompilerParams(needs_layout_passes=False),
        scratch_types=[
            pltpu.VMEM((N_NODES,), jnp.int32),
            pltpu.VMEM((VPW,), jnp.int32),
            pltpu.VMEM((VPW,), jnp.int32),
            pltpu.VMEM((VPW,), jnp.int32),
            pltpu.VMEM((VPW,), jnp.int32),
            pltpu.VMEM((VPW,), jnp.int32),
        ],
    )(S, src, dst)


def _sc_hist_body(att_hbm, sidx_hbm, bmax_hbm, out_hbm,
                  att_v, si_v, vv, iv2, bm_v, zb_v, B_sh):
    c = lax.axis_index("c")
    s = lax.axis_index("s")
    base = (s * NC + c) * EPW

    def zbody(i, carry):
        zb_v[pl.ds(i * 16, 16)] = jnp.zeros((16,), jnp.float32)
        return carry

    lax.fori_loop(0, WPT // 16, zbody, 0)
    pltpu.sync_copy(zb_v, B_sh.at[pl.ds(s * WPT, WPT)])
    plsc.subcore_barrier()

    pltpu.sync_copy(att_hbm.at[pl.ds(base, EPW)], att_v.at[pl.ds(0, EPW)])
    pltpu.sync_copy(sidx_hbm.at[pl.ds(base, EPW)], si_v.at[pl.ds(0, EPW)])
    pltpu.sync_copy(bmax_hbm, bm_v)
    m = bm_v[pl.ds(0, 16)]
    for k in range(1, GE // 16):
        m = jnp.maximum(m, bm_v[pl.ds(k * 16, 16)])
    g = lax.reduce_max(m, axes=(0,))
    lane = lax.iota(jnp.int32, 16)

    def body(i, carry):
        p = i * 16
        valid = (p + lane) < EPW
        e = jnp.exp(att_v[pl.ds(p, 16)] - g)
        e = jnp.where(valid, e, 0.0)
        ix = si_v[pl.ds(p, 16)]
        ix = jnp.where(valid, ix, 0)
        ix = jnp.minimum(jnp.maximum(ix, 0), BSZ - 1)
        vv[pl.ds(p, 16)] = e
        iv2[i // 8, pl.ds((i % 8) * 16, 16)] = ix
        return carry

    lax.fori_loop(0, EPAD // 16, body, 0)

    def srow(j, carry):
        pltpu.sync_copy(vv.at[pl.ds(j * 128, 128)], B_sh.at[iv2.at[j]],
                        add=True)
        return carry

    lax.fori_loop(0, ROWS, srow, 0)
    plsc.subcore_barrier()

    pltpu.sync_copy(B_sh.at[pl.ds(s * WPT, WPT)], zb_v)
    pltpu.sync_copy(zb_v, out_hbm.at[pl.ds(c * BSZ + s * WPT, WPT)])


def _sc_hist(att, sidx, bmax):
    mesh = plsc.VectorSubcoreMesh(core_axis_name="c", subcore_axis_name="s")
    return pl.kernel(
        _sc_hist_body,
        out_type=[jax.ShapeDtypeStruct((NC * BSZ,), jnp.float32)],
        mesh=mesh,
        compiler_params=pltpu.CompilerParams(needs_layout_passes=False),
        scratch_types=[
            pltpu.VMEM((EPAD,), jnp.float32),
            pltpu.VMEM((EPAD,), jnp.int32),
            pltpu.VMEM((EPAD,), jnp.float32),
            pltpu.VMEM((ROWS, 128), jnp.int32),
            pltpu.VMEM((GE,), jnp.float32),
            pltpu.VMEM((WPT,), jnp.float32),
            pltpu.VMEM_SHARED((BSZ,), jnp.float32),
        ],
    )(att, sidx, bmax)


def _att_body(ea_ref, ss_ref, sd_ref, we_ref, ws_ref, wsb_ref, wdb_ref,
              a2_ref, att_ref, bmax_ref):
    f32 = jnp.float32
    bf = jnp.bfloat16
    ts = jnp.dot(ws_ref[...], wsb_ref[...],
                 preferred_element_type=f32).astype(bf)
    td = jnp.dot(ws_ref[...], wdb_ref[...],
                 preferred_element_type=f32).astype(bf)
    u = jnp.dot(ea_ref[...].astype(bf), we_ref[...].astype(bf),
                preferred_element_type=f32)
    iot = lax.broadcasted_iota(jnp.int32, (LB, 1), 0)
    ohsT = (ss_ref[0] == iot).astype(bf)
    ohdT = (sd_ref[0] == iot).astype(bf)
    ohb = jnp.concatenate([ohsT, ohdT], axis=0)
    tsd = jnp.concatenate([ts, td], axis=0)
    dn = (((0,), (0,)), ((), ()))
    z = u + lax.dot_general(ohb, tsd, dn, preferred_element_type=f32)
    zl = jnp.maximum(z, 0.2 * z)
    att = lax.dot_general(a2_ref[...], zl, (((1,), (1,)), ((), ())),
                          preferred_element_type=f32)
    att_ref[...] = att[None]
    bmax_ref[...] = jnp.broadcast_to(jnp.max(att), (1, 1, 1))


def _out_body(bp_ref, ws_ref, wsb_ref, wo_ref, b_ref, out_ref):
    f32 = jnp.float32
    ts = jnp.dot(ws_ref[...], wsb_ref[...], preferred_element_type=f32)
    m = jnp.dot(ts, wo_ref[...], preferred_element_type=f32)
    bs = bp_ref[0] + bp_ref[1]
    denom = jnp.sum(bs, axis=1, keepdims=True)
    a = bs / (denom + 1e-16)
    logits = jnp.dot(a, m, preferred_element_type=f32) + b_ref[...]
    col = lax.broadcasted_iota(jnp.int32, (1, 128), 1)
    valid = col < V
    lm = jnp.where(valid, logits, -1e30)
    mx = jnp.max(lm, axis=1, keepdims=True)
    ex = jnp.where(valid, jnp.exp(logits - mx), 0.0)
    lse = mx + jnp.log(jnp.sum(ex, axis=1, keepdims=True))
    out_ref[...] = (logits - lse)[:, :V]


def kernel(S, edge_index, edge_attr, W_src1, W_dst1, W_e1, a1, W_s,
           W_src2, W_dst2, W_e2, a2, W_out, b_out):
    f32 = jnp.float32
    S = S.astype(jnp.int32)
    src = edge_index[0].astype(jnp.int32)
    dst = edge_index[1].astype(jnp.int32)

    ws_pad = jnp.zeros((LB, D), f32).at[:V].set(W_s)
    wo_pad = jnp.zeros((D, 128), f32).at[:, :V].set(W_out)
    b_pad = jnp.zeros((1, 128), f32).at[0, :V].set(b_out)

    ssrc, sdst, sidx = _sc_gather(S, src, dst)

    att3, bmax3 = pl.pallas_call(
        _att_body,
        grid=(GE,),
        in_specs=[
            pl.BlockSpec((BE, 128), lambda i: (i, 0)),
            pl.BlockSpec((1, 1, BE), lambda i: (i, 0, 0)),
            pl.BlockSpec((1, 1, BE), lambda i: (i, 0, 0)),
            pl.BlockSpec((128, D), lambda i: (0, 0)),
            pl.BlockSpec((LB, D), lambda i: (0, 0)),
            pl.BlockSpec((D, D), lambda i: (0, 0)),
            pl.BlockSpec((D, D), lambda i: (0, 0)),
            pl.BlockSpec((1, D), lambda i: (0, 0)),
        ],
        out_specs=[
            pl.BlockSpec((1, 1, BE), lambda i: (i, 0, 0)),
            pl.BlockSpec((1, 1, 1), lambda i: (i, 0, 0)),
        ],
        out_shape=[
            jax.ShapeDtypeStruct((GE, 1, BE), f32),
            jax.ShapeDtypeStruct((GE, 1, 1), f32),
        ],
    )(edge_attr, ssrc.reshape(GE, 1, BE), sdst.reshape(GE, 1, BE),
      W_e2, ws_pad, W_src2[D:], W_dst2[D:], a2.reshape(1, D))

    (bp,) = _sc_hist(att3.reshape(N_EDGES), sidx, bmax3.reshape(GE))

    out = pl.pallas_call(
        _out_body,
        grid=(N_NODES // BN,),
        in_specs=[
            pl.BlockSpec((NC, BN, LB), lambda i: (0, i, 0)),
            pl.BlockSpec((LB, D), lambda i: (0, 0)),
            pl.BlockSpec((D, D), lambda i: (0, 0)),
            pl.BlockSpec((D, 128), lambda i: (0, 0)),
            pl.BlockSpec((1, 128), lambda i: (0, 0)),
        ],
        out_specs=pl.BlockSpec((BN, V), lambda i: (i, 0)),
        out_shape=jax.ShapeDtypeStruct((N_NODES, V), f32),
    )(bp.reshape(NC, N_NODES, LB), ws_pad, W_src2[D:], wo_pad, b_pad)

    return out


# speedup vs baseline: 2.4571x; 1.0488x over previous
import jax
import jax.numpy as jnp
from jax import lax
from jax.experimental import pallas as pl
from jax.experimental.pallas import tpu as pltpu
from jax.experimental.pallas import tpu_sc as plsc

N_NODES = 10000
N_EDGES = 160000
D = 256
V = 21
LB = 32
NC, NS = 2, 16
NW = NC * NS
EPW = N_EDGES // NW
VPW = (EPW + 15) // 16 * 16
ROWS = (EPW + 127) // 128
EPAD = ROWS * 128
BSZ = N_NODES * LB
WPT = BSZ // NS

BE = 10000
GE = N_EDGES // BE
BN = 2000


def _sc_gather_body(S_hbm, src_hbm, dst_hbm, ssrc_hbm, sdst_hbm, sidx_hbm,
                    S_v, src_v, dst_v, ssrc_v, sdst_v, sidx_v):
    c = lax.axis_index("c")
    s = lax.axis_index("s")
    base = (s * NC + c) * EPW
    pltpu.sync_copy(S_hbm, S_v)
    pltpu.sync_copy(src_hbm.at[pl.ds(base, EPW)], src_v.at[pl.ds(0, EPW)])
    pltpu.sync_copy(dst_hbm.at[pl.ds(base, EPW)], dst_v.at[pl.ds(0, EPW)])

    def body(i, carry):
        p = i * 16
        sv = src_v[pl.ds(p, 16)]
        dv = dst_v[pl.ds(p, 16)]
        sv = jnp.minimum(jnp.maximum(sv, 0), N_NODES - 1)
        dv = jnp.minimum(jnp.maximum(dv, 0), N_NODES - 1)
        a = plsc.load_gather(S_v, [sv])
        b = plsc.load_gather(S_v, [dv])
        ssrc_v[pl.ds(p, 16)] = a
        sdst_v[pl.ds(p, 16)] = b
        sidx_v[pl.ds(p, 16)] = dv * LB + a
        return carry

    lax.fori_loop(0, VPW // 16, body, 0)
    pltpu.sync_copy(ssrc_v.at[pl.ds(0, EPW)], ssrc_hbm.at[pl.ds(base, EPW)])
    pltpu.sync_copy(sdst_v.at[pl.ds(0, EPW)], sdst_hbm.at[pl.ds(base, EPW)])
    pltpu.sync_copy(sidx_v.at[pl.ds(0, EPW)], sidx_hbm.at[pl.ds(base, EPW)])


def _sc_gather(S, src, dst):
    mesh = plsc.VectorSubcoreMesh(core_axis_name="c", subcore_axis_name="s")
    return pl.kernel(
        _sc_gather_body,
        out_type=[jax.ShapeDtypeStruct((N_EDGES,), jnp.int32)] * 3,
        mesh=mesh,
        compiler_params=pltpu.CompilerParams(needs_layout_passes=False),
        scratch_types=[
            pltpu.VMEM((N_NODES,), jnp.int32),
            pltpu.VMEM((VPW,), jnp.int32),
            pltpu.VMEM((VPW,), jnp.int32),
            pltpu.VMEM((VPW,), jnp.int32),
            pltpu.VMEM((VPW,), jnp.int32),
            pltpu.VMEM((VPW,), jnp.int32),
        ],
    )(S, src, dst)


def _sc_hist_body(att_hbm, sidx_hbm, bmax_hbm, out_hbm,
                  att_v, si_v, vv, iv2, bm_v, zb_v, B_sh):
    c = lax.axis_index("c")
    s = lax.axis_index("s")
    base = (s * NC + c) * EPW

    def zbody(i, carry):
        zb_v[pl.ds(i * 16, 16)] = jnp.zeros((16,), jnp.float32)
        return carry

    lax.fori_loop(0, WPT // 16, zbody, 0)
    pltpu.sync_copy(zb_v, B_sh.at[pl.ds(s * WPT, WPT)])
    plsc.subcore_barrier()

    pltpu.sync_copy(att_hbm.at[pl.ds(base, EPW)], att_v.at[pl.ds(0, EPW)])
    pltpu.sync_copy(sidx_hbm.at[pl.ds(base, EPW)], si_v.at[pl.ds(0, EPW)])
    pltpu.sync_copy(bmax_hbm, bm_v)
    m = bm_v[pl.ds(0, 16)]
    for k in range(1, GE // 16):
        m = jnp.maximum(m, bm_v[pl.ds(k * 16, 16)])
    g = lax.reduce_max(m, axes=(0,))
    lane = lax.iota(jnp.int32, 16)

    def body(i, carry):
        p = i * 16
        valid = (p + lane) < EPW
        e = jnp.exp(att_v[pl.ds(p, 16)] - g)
        e = jnp.where(valid, e, 0.0)
        ix = si_v[pl.ds(p, 16)]
        ix = jnp.where(valid, ix, 0)
        ix = jnp.minimum(jnp.maximum(ix, 0), BSZ - 1)
        vv[pl.ds(p, 16)] = e
        iv2[i // 8, pl.ds((i % 8) * 16, 16)] = ix
        return carry

    lax.fori_loop(0, EPAD // 16, body, 0)

    def srow(j, carry):
        pltpu.sync_copy(vv.at[pl.ds(j * 128, 128)], B_sh.at[iv2.at[j]],
                        add=True)
        return carry

    lax.fori_loop(0, ROWS, srow, 0)
    plsc.subcore_barrier()

    pltpu.sync_copy(B_sh.at[pl.ds(s * WPT, WPT)], zb_v)
    pltpu.sync_copy(zb_v, out_hbm.at[pl.ds(c * BSZ + s * WPT, WPT)])


def _sc_hist(att, sidx, bmax):
    mesh = plsc.VectorSubcoreMesh(core_axis_name="c", subcore_axis_name="s")
    return pl.kernel(
        _sc_hist_body,
        out_type=[jax.ShapeDtypeStruct((NC * BSZ,), jnp.float32)],
        mesh=mesh,
        compiler_params=pltpu.CompilerParams(needs_layout_passes=False),
        scratch_types=[
            pltpu.VMEM((EPAD,), jnp.float32),
            pltpu.VMEM((EPAD,), jnp.int32),
            pltpu.VMEM((EPAD,), jnp.float32),
            pltpu.VMEM((ROWS, 128), jnp.int32),
            pltpu.VMEM((GE,), jnp.float32),
            pltpu.VMEM((WPT,), jnp.float32),
            pltpu.VMEM_SHARED((BSZ,), jnp.float32),
        ],
    )(att, sidx, bmax)


def _att_body(ea_ref, ss_ref, sd_ref, we_ref, ws_ref, wsb_ref, wdb_ref,
              a2_ref, att_ref, bmax_ref):
    f32 = jnp.float32
    bf = jnp.bfloat16
    ts = jnp.dot(ws_ref[...], wsb_ref[...],
                 preferred_element_type=f32).astype(bf)
    td = jnp.dot(ws_ref[...], wdb_ref[...],
                 preferred_element_type=f32).astype(bf)
    u = jnp.dot(ea_ref[...].astype(bf), we_ref[...].astype(bf),
                preferred_element_type=f32)
    iot = lax.broadcasted_iota(jnp.int32, (LB, 1), 0)
    ohsT = (ss_ref[0] == iot).astype(bf)
    ohdT = (sd_ref[0] == iot).astype(bf)
    ohb = jnp.concatenate([ohsT, ohdT], axis=0)
    tsd = jnp.concatenate([ts, td], axis=0)
    dn = (((0,), (0,)), ((), ()))
    z = u + lax.dot_general(ohb, tsd, dn, preferred_element_type=f32)
    zl = jnp.maximum(z, 0.2 * z).astype(bf)
    att = lax.dot_general(a2_ref[...].astype(bf), zl,
                          (((1,), (1,)), ((), ())),
                          preferred_element_type=f32)
    att_ref[...] = att[None]
    bmax_ref[...] = jnp.broadcast_to(jnp.max(att), (1, 1, 1))


def _out_body(bp_ref, ws_ref, wsb_ref, wo_ref, b_ref, out_ref):
    f32 = jnp.float32
    ts = jnp.dot(ws_ref[...], wsb_ref[...], preferred_element_type=f32)
    m = jnp.dot(ts, wo_ref[...], preferred_element_type=f32)
    bs = bp_ref[0] + bp_ref[1]
    denom = jnp.sum(bs, axis=1, keepdims=True)
    a = bs / (denom + 1e-16)
    logits = jnp.dot(a, m, preferred_element_type=f32) + b_ref[...]
    col = lax.broadcasted_iota(jnp.int32, (1, 128), 1)
    valid = col < V
    lm = jnp.where(valid, logits, -1e30)
    mx = jnp.max(lm, axis=1, keepdims=True)
    ex = jnp.where(valid, jnp.exp(logits - mx), 0.0)
    lse = mx + jnp.log(jnp.sum(ex, axis=1, keepdims=True))
    out_ref[...] = (logits - lse)[:, :V]


def kernel(S, edge_index, edge_attr, W_src1, W_dst1, W_e1, a1, W_s,
           W_src2, W_dst2, W_e2, a2, W_out, b_out):
    f32 = jnp.float32
    S = S.astype(jnp.int32)
    src = edge_index[0].astype(jnp.int32)
    dst = edge_index[1].astype(jnp.int32)

    ws_pad = jnp.zeros((LB, D), f32).at[:V].set(W_s)
    wo_pad = jnp.zeros((D, 128), f32).at[:, :V].set(W_out)
    b_pad = jnp.zeros((1, 128), f32).at[0, :V].set(b_out)

    ssrc, sdst, sidx = _sc_gather(S, src, dst)

    att3, bmax3 = pl.pallas_call(
        _att_body,
        grid=(GE,),
        in_specs=[
            pl.BlockSpec((BE, 128), lambda i: (i, 0)),
            pl.BlockSpec((1, 1, BE), lambda i: (i, 0, 0)),
            pl.BlockSpec((1, 1, BE), lambda i: (i, 0, 0)),
            pl.BlockSpec((128, D), lambda i: (0, 0)),
            pl.BlockSpec((LB, D), lambda i: (0, 0)),
            pl.BlockSpec((D, D), lambda i: (0, 0)),
            pl.BlockSpec((D, D), lambda i: (0, 0)),
            pl.BlockSpec((1, D), lambda i: (0, 0)),
        ],
        out_specs=[
            pl.BlockSpec((1, 1, BE), lambda i: (i, 0, 0)),
            pl.BlockSpec((1, 1, 1), lambda i: (i, 0, 0)),
        ],
        out_shape=[
            jax.ShapeDtypeStruct((GE, 1, BE), f32),
            jax.ShapeDtypeStruct((GE, 1, 1), f32),
        ],
    )(edge_attr, ssrc.reshape(GE, 1, BE), sdst.reshape(GE, 1, BE),
      W_e2, ws_pad, W_src2[D:], W_dst2[D:], a2.reshape(1, D))

    (bp,) = _sc_hist(att3.reshape(N_EDGES), sidx, bmax3.reshape(GE))

    out = pl.pallas_call(
        _out_body,
        grid=(N_NODES // BN,),
        in_specs=[
            pl.BlockSpec((NC, BN, LB), lambda i: (0, i, 0)),
            pl.BlockSpec((LB, D), lambda i: (0, 0)),
            pl.BlockSpec((D, D), lambda i: (0, 0)),
            pl.BlockSpec((D, 128), lambda i: (0, 0)),
            pl.BlockSpec((1, 128), lambda i: (0, 0)),
        ],
        out_specs=pl.BlockSpec((BN, V), lambda i: (i, 0)),
        out_shape=jax.ShapeDtypeStruct((N_NODES, V), f32),
    )(bp.reshape(NC, N_NODES, LB), ws_pad, W_src2[D:], wo_pad, b_pad)

    return out


# speedup vs baseline: 2.4788x; 1.0088x over previous
import jax
import jax.numpy as jnp
from jax import lax
from jax.experimental import pallas as pl
from jax.experimental.pallas import tpu as pltpu
from jax.experimental.pallas import tpu_sc as plsc

N_NODES = 10000
N_EDGES = 160000
D = 256
V = 21
LB = 32
NC, NS = 2, 16
NW = NC * NS
EPW = N_EDGES // NW
VPW = (EPW + 15) // 16 * 16
ROWS = (EPW + 127) // 128
EPAD = ROWS * 128
BSZ = N_NODES * LB
WPT = BSZ // NS

BE = 5000
GE = N_EDGES // BE
BN = 2000


def _sc_gather_body(S_hbm, src_hbm, dst_hbm, ssrc_hbm, sdst_hbm, sidx_hbm,
                    S_v, src_v, dst_v, ssrc_v, sdst_v, sidx_v):
    c = lax.axis_index("c")
    s = lax.axis_index("s")
    w = s * NC + c
    base = w * EPW
    pltpu.sync_copy(S_hbm, S_v)
    pltpu.sync_copy(src_hbm.at[pl.ds(base, EPW)], src_v.at[pl.ds(0, EPW)])
    pltpu.sync_copy(dst_hbm.at[pl.ds(base, EPW)], dst_v.at[pl.ds(0, EPW)])

    def body(i, carry):
        p = i * 16
        sv = src_v[pl.ds(p, 16)]
        dv = dst_v[pl.ds(p, 16)]
        sv = jnp.minimum(jnp.maximum(sv, 0), N_NODES - 1)
        dv = jnp.minimum(jnp.maximum(dv, 0), N_NODES - 1)
        a = plsc.load_gather(S_v, [sv])
        b = plsc.load_gather(S_v, [dv])
        ssrc_v[pl.ds(p, 16)] = a
        sdst_v[pl.ds(p, 16)] = b
        sidx_v[pl.ds(p, 16)] = dv * LB + a
        return carry

    lax.fori_loop(0, VPW // 16, body, 0)
    pltpu.sync_copy(ssrc_v.at[pl.ds(0, EPW)], ssrc_hbm.at[w, 0, pl.ds(0, EPW)])
    pltpu.sync_copy(sdst_v.at[pl.ds(0, EPW)], sdst_hbm.at[w, 0, pl.ds(0, EPW)])
    pltpu.sync_copy(sidx_v.at[pl.ds(0, EPW)], sidx_hbm.at[pl.ds(base, EPW)])


def _sc_gather(S, src, dst):
    mesh = plsc.VectorSubcoreMesh(core_axis_name="c", subcore_axis_name="s")
    return pl.kernel(
        _sc_gather_body,
        out_type=[jax.ShapeDtypeStruct((GE, 1, BE), jnp.int32),
                  jax.ShapeDtypeStruct((GE, 1, BE), jnp.int32),
                  jax.ShapeDtypeStruct((N_EDGES,), jnp.int32)],
        mesh=mesh,
        compiler_params=pltpu.CompilerParams(needs_layout_passes=False),
        scratch_types=[
            pltpu.VMEM((N_NODES,), jnp.int32),
            pltpu.VMEM((VPW,), jnp.int32),
            pltpu.VMEM((VPW,), jnp.int32),
            pltpu.VMEM((VPW,), jnp.int32),
            pltpu.VMEM((VPW,), jnp.int32),
            pltpu.VMEM((VPW,), jnp.int32),
        ],
    )(S, src, dst)


def _sc_hist_body(att_hbm, sidx_hbm, bmax_hbm, out_hbm,
                  att_v, si_v, vv, iv2, bm_v, zb_v, B_sh):
    c = lax.axis_index("c")
    s = lax.axis_index("s")
    w = s * NC + c
    base = w * EPW

    def zbody(i, carry):
        zb_v[pl.ds(i * 16, 16)] = jnp.zeros((16,), jnp.float32)
        return carry

    lax.fori_loop(0, WPT // 16, zbody, 0)
    pltpu.sync_copy(zb_v, B_sh.at[pl.ds(s * WPT, WPT)])
    plsc.subcore_barrier()

    pltpu.sync_copy(att_hbm.at[w, 0, pl.ds(0, EPW)], att_v.at[pl.ds(0, EPW)])
    pltpu.sync_copy(sidx_hbm.at[pl.ds(base, EPW)], si_v.at[pl.ds(0, EPW)])
    pltpu.sync_copy(bmax_hbm, bm_v)
    m = bm_v[pl.ds(0, 16)]
    for k in range(1, GE // 16):
        m = jnp.maximum(m, bm_v[pl.ds(k * 16, 16)])
    g = lax.reduce_max(m, axes=(0,))
    lane = lax.iota(jnp.int32, 16)

    def body(i, carry):
        p = i * 16
        valid = (p + lane) < EPW
        e = jnp.exp(att_v[pl.ds(p, 16)] - g)
        e = jnp.where(valid, e, 0.0)
        ix = si_v[pl.ds(p, 16)]
        ix = jnp.where(valid, ix, 0)
        ix = jnp.minimum(jnp.maximum(ix, 0), BSZ - 1)
        vv[pl.ds(p, 16)] = e
        iv2[i // 8, pl.ds((i % 8) * 16, 16)] = ix
        return carry

    lax.fori_loop(0, EPAD // 16, body, 0)

    def srow(j, carry):
        pltpu.sync_copy(vv.at[pl.ds(j * 128, 128)], B_sh.at[iv2.at[j]],
                        add=True)
        return carry

    lax.fori_loop(0, ROWS, srow, 0)
    plsc.subcore_barrier()

    pltpu.sync_copy(B_sh.at[pl.ds(s * WPT, WPT)], zb_v)
    pltpu.sync_copy(zb_v, out_hbm.at[pl.ds(c * BSZ + s * WPT, WPT)])


def _sc_hist(att, sidx, bmax):
    mesh = plsc.VectorSubcoreMesh(core_axis_name="c", subcore_axis_name="s")
    return pl.kernel(
        _sc_hist_body,
        out_type=[jax.ShapeDtypeStruct((NC * BSZ,), jnp.float32)],
        mesh=mesh,
        compiler_params=pltpu.CompilerParams(needs_layout_passes=False),
        scratch_types=[
            pltpu.VMEM((EPAD,), jnp.float32),
            pltpu.VMEM((EPAD,), jnp.int32),
            pltpu.VMEM((EPAD,), jnp.float32),
            pltpu.VMEM((ROWS, 128), jnp.int32),
            pltpu.VMEM((GE,), jnp.float32),
            pltpu.VMEM((WPT,), jnp.float32),
            pltpu.VMEM_SHARED((BSZ,), jnp.float32),
        ],
    )(att, sidx, bmax)


def _att_body(ea_ref, ss_ref, sd_ref, we_ref, ws_ref, wsb_ref, wdb_ref,
              a2_ref, att_ref, bmax_ref):
    f32 = jnp.float32
    bf = jnp.bfloat16
    ts = jnp.dot(ws_ref[...], wsb_ref[...],
                 preferred_element_type=f32).astype(bf)
    td = jnp.dot(ws_ref[...], wdb_ref[...],
                 preferred_element_type=f32).astype(bf)
    u = jnp.dot(ea_ref[...].astype(bf), we_ref[...].astype(bf),
                preferred_element_type=f32)
    iot = lax.broadcasted_iota(jnp.int32, (LB, 1), 0)
    ohsT = (ss_ref[0] == iot).astype(bf)
    ohdT = (sd_ref[0] == iot).astype(bf)
    ohb = jnp.concatenate([ohsT, ohdT], axis=0)
    tsd = jnp.concatenate([ts, td], axis=0)
    dn = (((0,), (0,)), ((), ()))
    z = u + lax.dot_general(ohb, tsd, dn, preferred_element_type=f32)
    zl = jnp.maximum(z, 0.2 * z).astype(bf)
    att = lax.dot_general(a2_ref[...].astype(bf), zl,
                          (((1,), (1,)), ((), ())),
                          preferred_element_type=f32)
    att_ref[...] = att[None]
    bmax_ref[...] = jnp.broadcast_to(jnp.max(att), (1, 1, 1))


def _out_body(bp_ref, ws_ref, wsb_ref, wo_ref, b_ref, out_ref):
    f32 = jnp.float32
    ts = jnp.dot(ws_ref[...], wsb_ref[...], preferred_element_type=f32)
    m = jnp.dot(ts, wo_ref[...], preferred_element_type=f32)
    bs = bp_ref[0] + bp_ref[1]
    denom = jnp.sum(bs, axis=1, keepdims=True)
    a = bs / (denom + 1e-16)
    logits = jnp.dot(a, m, preferred_element_type=f32) + b_ref[...]
    col = lax.broadcasted_iota(jnp.int32, (1, 128), 1)
    valid = col < V
    lm = jnp.where(valid, logits, -1e30)
    mx = jnp.max(lm, axis=1, keepdims=True)
    ex = jnp.where(valid, jnp.exp(logits - mx), 0.0)
    lse = mx + jnp.log(jnp.sum(ex, axis=1, keepdims=True))
    out_ref[...] = (logits - lse)[:, :V]


def kernel(S, edge_index, edge_attr, W_src1, W_dst1, W_e1, a1, W_s,
           W_src2, W_dst2, W_e2, a2, W_out, b_out):
    f32 = jnp.float32
    S = S.astype(jnp.int32)
    src = edge_index[0].astype(jnp.int32)
    dst = edge_index[1].astype(jnp.int32)

    ws_pad = jnp.zeros((LB, D), f32).at[:V].set(W_s)
    wo_pad = jnp.zeros((D, 128), f32).at[:, :V].set(W_out)
    b_pad = jnp.zeros((1, 128), f32).at[0, :V].set(b_out)

    ssrc3, sdst3, sidx = _sc_gather(S, src, dst)

    att3, bmax3 = pl.pallas_call(
        _att_body,
        grid=(GE,),
        in_specs=[
            pl.BlockSpec((BE, 128), lambda i: (i, 0)),
            pl.BlockSpec((1, 1, BE), lambda i: (i, 0, 0)),
            pl.BlockSpec((1, 1, BE), lambda i: (i, 0, 0)),
            pl.BlockSpec((128, D), lambda i: (0, 0)),
            pl.BlockSpec((LB, D), lambda i: (0, 0)),
            pl.BlockSpec((D, D), lambda i: (0, 0)),
            pl.BlockSpec((D, D), lambda i: (0, 0)),
            pl.BlockSpec((1, D), lambda i: (0, 0)),
        ],
        out_specs=[
            pl.BlockSpec((1, 1, BE), lambda i: (i, 0, 0)),
            pl.BlockSpec((1, 1, 1), lambda i: (i, 0, 0)),
        ],
        out_shape=[
            jax.ShapeDtypeStruct((GE, 1, BE), f32),
            jax.ShapeDtypeStruct((GE, 1, 1), f32),
        ],
    )(edge_attr, ssrc3, sdst3,
      W_e2, ws_pad, W_src2[D:], W_dst2[D:], a2.reshape(1, D))

    (bp,) = _sc_hist(att3, sidx, bmax3.reshape(GE))

    out = pl.pallas_call(
        _out_body,
        grid=(N_NODES // BN,),
        in_specs=[
            pl.BlockSpec((NC, BN, LB), lambda i: (0, i, 0)),
            pl.BlockSpec((LB, D), lambda i: (0, 0)),
            pl.BlockSpec((D, D), lambda i: (0, 0)),
            pl.BlockSpec((D, 128), lambda i: (0, 0)),
            pl.BlockSpec((1, 128), lambda i: (0, 0)),
        ],
        out_specs=pl.BlockSpec((BN, V), lambda i: (i, 0)),
        out_shape=jax.ShapeDtypeStruct((N_NODES, V), f32),
    )(bp.reshape(NC, N_NODES, LB), ws_pad, W_src2[D:], wo_pad, b_pad)

    return out


# speedup vs baseline: 2.5140x; 1.0142x over previous
import jax
import jax.numpy as jnp
from jax import lax
from jax.experimental import pallas as pl
from jax.experimental.pallas import tpu as pltpu
from jax.experimental.pallas import tpu_sc as plsc

N_NODES = 10000
N_EDGES = 160000
D = 256
V = 21
LB = 32
NC, NS = 2, 16
NW = NC * NS
EPW = N_EDGES // NW
VPW = (EPW + 15) // 16 * 16
ROWS = (EPW + 127) // 128
EPAD = ROWS * 128
BSZ = N_NODES * LB
WPT = BSZ // NS

BE = 5000
GE = N_EDGES // BE
BN = 2000


def _sc_gather_body(S_hbm, src_hbm, dst_hbm, ssrc_hbm, sdst_hbm, sidx_hbm,
                    S_v, src_v, dst_v, ssrc_v, sdst_v, sidx_v):
    c = lax.axis_index("c")
    s = lax.axis_index("s")
    w = s * NC + c
    base = w * EPW
    pltpu.sync_copy(S_hbm, S_v)
    pltpu.sync_copy(src_hbm.at[pl.ds(base, EPW)], src_v.at[pl.ds(0, EPW)])
    pltpu.sync_copy(dst_hbm.at[pl.ds(base, EPW)], dst_v.at[pl.ds(0, EPW)])

    def body(i, carry):
        p = i * 16
        sv = src_v[pl.ds(p, 16)]
        dv = dst_v[pl.ds(p, 16)]
        sv = jnp.minimum(jnp.maximum(sv, 0), N_NODES - 1)
        dv = jnp.minimum(jnp.maximum(dv, 0), N_NODES - 1)
        a = plsc.load_gather(S_v, [sv])
        b = plsc.load_gather(S_v, [dv])
        ssrc_v[pl.ds(p, 16)] = a
        sdst_v[pl.ds(p, 16)] = b
        sidx_v[pl.ds(p, 16)] = dv * LB + a
        return carry

    lax.fori_loop(0, VPW // 16, body, 0)
    pltpu.sync_copy(ssrc_v.at[pl.ds(0, EPW)], ssrc_hbm.at[w, 0, pl.ds(0, EPW)])
    pltpu.sync_copy(sdst_v.at[pl.ds(0, EPW)], sdst_hbm.at[w, 0, pl.ds(0, EPW)])
    pltpu.sync_copy(sidx_v.at[pl.ds(0, EPW)], sidx_hbm.at[pl.ds(base, EPW)])


def _sc_gather(S, src, dst):
    mesh = plsc.VectorSubcoreMesh(core_axis_name="c", subcore_axis_name="s")
    return pl.kernel(
        _sc_gather_body,
        out_type=[jax.ShapeDtypeStruct((GE, 1, BE), jnp.int32),
                  jax.ShapeDtypeStruct((GE, 1, BE), jnp.int32),
                  jax.ShapeDtypeStruct((N_EDGES,), jnp.int32)],
        mesh=mesh,
        compiler_params=pltpu.CompilerParams(needs_layout_passes=False),
        scratch_types=[
            pltpu.VMEM((N_NODES,), jnp.int32),
            pltpu.VMEM((VPW,), jnp.int32),
            pltpu.VMEM((VPW,), jnp.int32),
            pltpu.VMEM((VPW,), jnp.int32),
            pltpu.VMEM((VPW,), jnp.int32),
            pltpu.VMEM((VPW,), jnp.int32),
        ],
    )(S, src, dst)


def _sc_hist_body(att_hbm, sidx_hbm, bmax_hbm, out_hbm,
                  att_v, si_v, vv, iv2, bm_v, zb_v, B_sh, ldsem):
    c = lax.axis_index("c")
    s = lax.axis_index("s")
    w = s * NC + c
    base = w * EPW

    cp1 = pltpu.async_copy(att_hbm.at[w, 0, pl.ds(0, EPW)],
                           att_v.at[pl.ds(0, EPW)], ldsem)
    cp2 = pltpu.async_copy(sidx_hbm.at[pl.ds(base, EPW)],
                           si_v.at[pl.ds(0, EPW)], ldsem)
    cp3 = pltpu.async_copy(bmax_hbm, bm_v, ldsem)

    def zbody(i, carry):
        zb_v[pl.ds(i * 16, 16)] = jnp.zeros((16,), jnp.float32)
        return carry

    lax.fori_loop(0, WPT // 16, zbody, 0)
    pltpu.sync_copy(zb_v, B_sh.at[pl.ds(s * WPT, WPT)])
    plsc.subcore_barrier()

    cp1.wait()
    cp2.wait()
    cp3.wait()
    m = bm_v[pl.ds(0, 16)]
    for k in range(1, GE // 16):
        m = jnp.maximum(m, bm_v[pl.ds(k * 16, 16)])
    g = lax.reduce_max(m, axes=(0,))
    lane = lax.iota(jnp.int32, 16)

    def body(i, carry):
        p = i * 16
        valid = (p + lane) < EPW
        e = jnp.exp(att_v[pl.ds(p, 16)] - g)
        e = jnp.where(valid, e, 0.0)
        ix = si_v[pl.ds(p, 16)]
        ix = jnp.where(valid, ix, 0)
        ix = jnp.minimum(jnp.maximum(ix, 0), BSZ - 1)
        vv[pl.ds(p, 16)] = e
        iv2[i // 8, pl.ds((i % 8) * 16, 16)] = ix
        return carry

    lax.fori_loop(0, EPAD // 16, body, 0)

    def srow(j, carry):
        pltpu.sync_copy(vv.at[pl.ds(j * 128, 128)], B_sh.at[iv2.at[j]],
                        add=True)
        return carry

    lax.fori_loop(0, ROWS, srow, 0)
    plsc.subcore_barrier()

    pltpu.sync_copy(B_sh.at[pl.ds(s * WPT, WPT)], zb_v)
    pltpu.sync_copy(zb_v, out_hbm.at[pl.ds(c * BSZ + s * WPT, WPT)])


def _sc_hist(att, sidx, bmax):
    mesh = plsc.VectorSubcoreMesh(core_axis_name="c", subcore_axis_name="s")
    return pl.kernel(
        _sc_hist_body,
        out_type=[jax.ShapeDtypeStruct((NC * BSZ,), jnp.float32)],
        mesh=mesh,
        compiler_params=pltpu.CompilerParams(needs_layout_passes=False),
        scratch_types=[
            pltpu.VMEM((EPAD,), jnp.float32),
            pltpu.VMEM((EPAD,), jnp.int32),
            pltpu.VMEM((EPAD,), jnp.float32),
            pltpu.VMEM((ROWS, 128), jnp.int32),
            pltpu.VMEM((GE,), jnp.float32),
            pltpu.VMEM((WPT,), jnp.float32),
            pltpu.VMEM_SHARED((BSZ,), jnp.float32),
            pltpu.SemaphoreType.DMA,
        ],
    )(att, sidx, bmax)


def _att_body(ea_ref, ss_ref, sd_ref, we_ref, ws_ref, wsb_ref, wdb_ref,
              a2_ref, att_ref, bmax_ref):
    f32 = jnp.float32
    bf = jnp.bfloat16
    ts = jnp.dot(ws_ref[...], wsb_ref[...],
                 preferred_element_type=f32).astype(bf)
    td = jnp.dot(ws_ref[...], wdb_ref[...],
                 preferred_element_type=f32).astype(bf)
    u = jnp.dot(ea_ref[...].astype(bf), we_ref[...].astype(bf),
                preferred_element_type=f32)
    iot = lax.broadcasted_iota(jnp.int32, (LB, 1), 0)
    ohsT = (ss_ref[0] == iot).astype(bf)
    ohdT = (sd_ref[0] == iot).astype(bf)
    ohb = jnp.concatenate([ohsT, ohdT], axis=0)
    tsd = jnp.concatenate([ts, td], axis=0)
    dn = (((0,), (0,)), ((), ()))
    z = u + lax.dot_general(ohb, tsd, dn, preferred_element_type=f32)
    zl = jnp.maximum(z, 0.2 * z).astype(bf)
    att = lax.dot_general(a2_ref[...].astype(bf), zl,
                          (((1,), (1,)), ((), ())),
                          preferred_element_type=f32)
    att_ref[...] = att[None]
    bmax_ref[...] = jnp.broadcast_to(jnp.max(att), (1, 1, 1))


def _out_body(bp_ref, ws_ref, wsb_ref, wo_ref, b_ref, out_ref):
    f32 = jnp.float32
    ts = jnp.dot(ws_ref[...], wsb_ref[...], preferred_element_type=f32)
    m = jnp.dot(ts, wo_ref[...], preferred_element_type=f32)
    bs = bp_ref[0] + bp_ref[1]
    denom = jnp.sum(bs, axis=1, keepdims=True)
    a = bs / (denom + 1e-16)
    logits = jnp.dot(a, m, preferred_element_type=f32) + b_ref[...]
    col = lax.broadcasted_iota(jnp.int32, (1, 128), 1)
    valid = col < V
    lm = jnp.where(valid, logits, -1e30)
    mx = jnp.max(lm, axis=1, keepdims=True)
    ex = jnp.where(valid, jnp.exp(logits - mx), 0.0)
    lse = mx + jnp.log(jnp.sum(ex, axis=1, keepdims=True))
    out_ref[...] = (logits - lse)[:, :V]


def kernel(S, edge_index, edge_attr, W_src1, W_dst1, W_e1, a1, W_s,
           W_src2, W_dst2, W_e2, a2, W_out, b_out):
    f32 = jnp.float32
    S = S.astype(jnp.int32)
    src = edge_index[0].astype(jnp.int32)
    dst = edge_index[1].astype(jnp.int32)

    ws_pad = jnp.zeros((LB, D), f32).at[:V].set(W_s)
    wo_pad = jnp.zeros((D, 128), f32).at[:, :V].set(W_out)
    b_pad = jnp.zeros((1, 128), f32).at[0, :V].set(b_out)

    ssrc3, sdst3, sidx = _sc_gather(S, src, dst)

    att3, bmax3 = pl.pallas_call(
        _att_body,
        grid=(GE,),
        in_specs=[
            pl.BlockSpec((BE, 128), lambda i: (i, 0)),
            pl.BlockSpec((1, 1, BE), lambda i: (i, 0, 0)),
            pl.BlockSpec((1, 1, BE), lambda i: (i, 0, 0)),
            pl.BlockSpec((128, D), lambda i: (0, 0)),
            pl.BlockSpec((LB, D), lambda i: (0, 0)),
            pl.BlockSpec((D, D), lambda i: (0, 0)),
            pl.BlockSpec((D, D), lambda i: (0, 0)),
            pl.BlockSpec((1, D), lambda i: (0, 0)),
        ],
        out_specs=[
            pl.BlockSpec((1, 1, BE), lambda i: (i, 0, 0)),
            pl.BlockSpec((1, 1, 1), lambda i: (i, 0, 0)),
        ],
        out_shape=[
            jax.ShapeDtypeStruct((GE, 1, BE), f32),
            jax.ShapeDtypeStruct((GE, 1, 1), f32),
        ],
    )(edge_attr, ssrc3, sdst3,
      W_e2, ws_pad, W_src2[D:], W_dst2[D:], a2.reshape(1, D))

    (bp,) = _sc_hist(att3, sidx, bmax3.reshape(GE))

    out = pl.pallas_call(
        _out_body,
        grid=(N_NODES // BN,),
        in_specs=[
            pl.BlockSpec((NC, BN, LB), lambda i: (0, i, 0)),
            pl.BlockSpec((LB, D), lambda i: (0, 0)),
            pl.BlockSpec((D, D), lambda i: (0, 0)),
            pl.BlockSpec((D, 128), lambda i: (0, 0)),
            pl.BlockSpec((1, 128), lambda i: (0, 0)),
        ],
        out_specs=pl.BlockSpec((BN, V), lambda i: (i, 0)),
        out_shape=jax.ShapeDtypeStruct((N_NODES, V), f32),
    )(bp.reshape(NC, N_NODES, LB), ws_pad, W_src2[D:], wo_pad, b_pad)

    return out
